# Initial kernel scaffold; baseline (speedup 1.0000x reference)
import jax
import jax.numpy as jnp
from jax.experimental import pallas as pl


def kernel(z, codebooks):
    raise NotImplementedError("write your pallas kernel here")


# speedup vs baseline: 3.9350x; 3.9350x over previous
import functools

import jax
import jax.numpy as jnp
from jax.experimental import pallas as pl

B, D, H, W = 16, 384, 32, 32
N_CODEBOOKS, VOCAB, CDIM = 12, 1024, 32
HW = H * W


def _vq_kernel(z_ref, cb_ref, quant_ref, idx_ref, loss_ref):
    i = pl.program_id(0)
    b = pl.program_id(1)

    zb = z_ref[0]
    cb = cb_ref[0]

    c2 = jnp.sum(cb * cb, axis=1, keepdims=True)

    cz = jax.lax.dot_general(
        cb, zb,
        dimension_numbers=(((1,), (0,)), ((), ())),
        preferred_element_type=jnp.float32,
    )
    scores = c2 - 2.0 * cz

    idx = jnp.argmin(scores, axis=0).astype(jnp.int32)
    idx_ref[0, 0, 0] = idx

    onehot = (jax.lax.broadcasted_iota(jnp.int32, (VOCAB, HW), 0)
              == idx[None, :]).astype(jnp.float32)
    quant = jax.lax.dot_general(
        cb, onehot,
        dimension_numbers=(((0,), (0,)), ((), ())),
        preferred_element_type=jnp.float32,
    )
    quant_ref[0] = quant

    partial = jnp.sum((zb - quant) ** 2).reshape(1, 1)

    @pl.when(jnp.logical_and(i == 0, b == 0))
    def _init():
        loss_ref[...] = jnp.zeros_like(loss_ref)

    loss_ref[...] += partial


@functools.partial(jax.jit, static_argnames=())
def kernel(z, codebooks):
    z3 = z.reshape(B, D, HW)

    quant3, idx4, loss = pl.pallas_call(
        _vq_kernel,
        grid=(N_CODEBOOKS, B),
        in_specs=[
            pl.BlockSpec((1, CDIM, HW), lambda i, b: (b, i, 0)),
            pl.BlockSpec((1, VOCAB, CDIM), lambda i, b: (i, 0, 0)),
        ],
        out_specs=[
            pl.BlockSpec((1, CDIM, HW), lambda i, b: (b, i, 0)),
            pl.BlockSpec((1, 1, 1, HW), lambda i, b: (b, i, 0, 0)),
            pl.BlockSpec((1, 1), lambda i, b: (0, 0)),
        ],
        out_shape=[
            jax.ShapeDtypeStruct((B, D, HW), jnp.float32),
            jax.ShapeDtypeStruct((B, N_CODEBOOKS, 1, HW), jnp.int32),
            jax.ShapeDtypeStruct((1, 1), jnp.float32),
        ],
    )(z3, codebooks)

    quantized = quant3.reshape(B, D, H, W)
    indices_out = idx4.reshape(B, N_CODEBOOKS, H, W)
    commitment_loss = (loss[0, 0] / (B * HW * CDIM * N_CODEBOOKS)).astype(jnp.float32)
    return quantized, indices_out, commitment_loss


# speedup vs baseline: 5.1503x; 1.3089x over previous
import functools

import jax
import jax.numpy as jnp
from jax.experimental import pallas as pl
from jax.experimental.pallas import tpu as pltpu

B, D, H, W = 16, 384, 32, 32
N_CODEBOOKS, VOCAB, CDIM = 12, 1024, 32
HW = H * W
BB = 16


def _vq_kernel(z_ref, cb_ref, quant_ref, idx_ref, loss_ref, cba_ref, c2_ref,
               cbb_ref):
    i = pl.program_id(0)
    b2 = pl.program_id(1)

    @pl.when(b2 == 0)
    def _prep():
        cb = cb_ref[0]
        cba_ref[...] = -2.0 * cb
        c2_ref[...] = jnp.sum(cb * cb, axis=1, keepdims=True)
        cbb_ref[...] = cb.astype(jnp.bfloat16)

    cb_m2 = cba_ref[...]
    c2 = c2_ref[...]
    cb_bf = cbb_ref[...]

    total = jnp.zeros((1, 1), jnp.float32)
    for s in range(BB):
        zb = z_ref[s]
        scores = c2 + jax.lax.dot_general(
            cb_m2, zb,
            dimension_numbers=(((1,), (0,)), ((), ())),
            preferred_element_type=jnp.float32,
        )

        idx = jnp.argmin(scores, axis=0).astype(jnp.int32)
        idx_ref[s, 0, 0] = idx

        onehot = (jax.lax.broadcasted_iota(jnp.int32, (VOCAB, HW), 0)
                  == idx[None, :]).astype(jnp.bfloat16)
        quant = jax.lax.dot_general(
            cb_bf, onehot,
            dimension_numbers=(((0,), (0,)), ((), ())),
            preferred_element_type=jnp.float32,
        )
        quant_ref[s] = quant

        total = total + jnp.sum((zb - quant) ** 2).reshape(1, 1)

    @pl.when(jnp.logical_and(i == 0, b2 == 0))
    def _init():
        loss_ref[...] = jnp.zeros_like(loss_ref)

    loss_ref[...] += total


@functools.partial(jax.jit, static_argnames=())
def kernel(z, codebooks):
    z3 = z.reshape(B, D, HW)

    quant3, idx4, loss = pl.pallas_call(
        _vq_kernel,
        grid=(N_CODEBOOKS, B // BB),
        in_specs=[
            pl.BlockSpec((BB, CDIM, HW), lambda i, b: (b, i, 0)),
            pl.BlockSpec((1, VOCAB, CDIM), lambda i, b: (i, 0, 0)),
        ],
        out_specs=[
            pl.BlockSpec((BB, CDIM, HW), lambda i, b: (b, i, 0)),
            pl.BlockSpec((BB, 1, 1, HW), lambda i, b: (b, i, 0, 0)),
            pl.BlockSpec((1, 1), lambda i, b: (0, 0)),
        ],
        out_shape=[
            jax.ShapeDtypeStruct((B, D, HW), jnp.float32),
            jax.ShapeDtypeStruct((B, N_CODEBOOKS, 1, HW), jnp.int32),
            jax.ShapeDtypeStruct((1, 1), jnp.float32),
        ],
        scratch_shapes=[
            pltpu.VMEM((VOCAB, CDIM), jnp.float32),
            pltpu.VMEM((VOCAB, 1), jnp.float32),
            pltpu.VMEM((VOCAB, CDIM), jnp.bfloat16),
        ],
    )(z3, codebooks)

    quantized = quant3.reshape(B, D, H, W)
    indices_out = idx4.reshape(B, N_CODEBOOKS, H, W)
    commitment_loss = (loss[0, 0] / (B * HW * CDIM * N_CODEBOOKS)).astype(jnp.float32)
    return quantized, indices_out, commitment_loss


# speedup vs baseline: 5.8256x; 1.1311x over previous
import functools

import jax
import jax.numpy as jnp
from jax.experimental import pallas as pl
from jax.experimental.pallas import tpu as pltpu

B, D, H, W = 16, 384, 32, 32
N_CODEBOOKS, VOCAB, CDIM = 12, 1024, 32
HW = H * W
BB = 16
NL = 8
NH = VOCAB // NL


def _vq_kernel(z_ref, cb_ref, cbr_ref, quant_ref, idx_ref, loss_ref):
    i = pl.program_id(0)

    cb = cb_ref[0]
    cb_m2 = -2.0 * cb
    c2 = jnp.sum(cb * cb, axis=1, keepdims=True)
    cb_grp = cbr_ref[0].astype(jnp.bfloat16)

    total = jnp.zeros((1, 1), jnp.float32)
    for s in range(BB):
        zb = z_ref[s]
        scores = c2 + jax.lax.dot_general(
            cb_m2, zb,
            dimension_numbers=(((1,), (0,)), ((), ())),
            preferred_element_type=jnp.float32,
        )

        idx = jnp.argmin(scores, axis=0).astype(jnp.int32)
        idx_ref[s, 0, 0] = idx

        hi = jax.lax.shift_right_logical(idx, 3)
        lo = jnp.bitwise_and(idx, 7)
        ghot = (jax.lax.broadcasted_iota(jnp.int32, (NH, HW), 0)
                == hi[None, :]).astype(jnp.bfloat16)
        cand = jax.lax.dot_general(
            cb_grp, ghot,
            dimension_numbers=(((0,), (0,)), ((), ())),
            preferred_element_type=jnp.float32,
        )
        quant = jnp.zeros((CDIM, HW), jnp.float32)
        for l in range(NL):
            sel = (lo[None, :] == l).astype(jnp.float32)
            quant = quant + cand[l * CDIM:(l + 1) * CDIM] * sel
        quant_ref[s] = quant

        total = total + jnp.sum((zb - quant) ** 2).reshape(1, 1)

    @pl.when(i == 0)
    def _init():
        loss_ref[...] = jnp.zeros_like(loss_ref)

    loss_ref[...] += total


@functools.partial(jax.jit, static_argnames=())
def kernel(z, codebooks):
    z3 = z.reshape(B, D, HW)
    cbr = codebooks.reshape(N_CODEBOOKS, NH, NL * CDIM)

    quant3, idx4, loss = pl.pallas_call(
        _vq_kernel,
        grid=(N_CODEBOOKS,),
        in_specs=[
            pl.BlockSpec((BB, CDIM, HW), lambda i: (0, i, 0)),
            pl.BlockSpec((1, VOCAB, CDIM), lambda i: (i, 0, 0)),
            pl.BlockSpec((1, NH, NL * CDIM), lambda i: (i, 0, 0)),
        ],
        out_specs=[
            pl.BlockSpec((BB, CDIM, HW), lambda i: (0, i, 0)),
            pl.BlockSpec((BB, 1, 1, HW), lambda i: (0, i, 0, 0)),
            pl.BlockSpec((1, 1), lambda i: (0, 0)),
        ],
        out_shape=[
            jax.ShapeDtypeStruct((B, D, HW), jnp.float32),
            jax.ShapeDtypeStruct((B, N_CODEBOOKS, 1, HW), jnp.int32),
            jax.ShapeDtypeStruct((1, 1), jnp.float32),
        ],
    )(z3, codebooks, cbr)

    quantized = quant3.reshape(B, D, H, W)
    indices_out = idx4.reshape(B, N_CODEBOOKS, H, W)
    commitment_loss = (loss[0, 0] / (B * HW * CDIM * N_CODEBOOKS)).astype(jnp.float32)
    return quantized, indices_out, commitment_loss
